# baseline (speedup 1.0000x reference)
import jax
import jax.numpy as jnp
from jax import lax
from jax.experimental import pallas as pl
from jax.experimental.pallas import tpu as pltpu
from jax.experimental.pallas import tpu_sc as plsc

D_MODEL = 256
B, L = 16, 2048
NTOK = B * L
DM = D_MODEL // 2

NC, NS, LANES = 2, 16, 16
NW = NC * NS
TPW = NTOK // NW
CHUNK = 64
NCHUNK = TPW // CHUNK
NBUF = 3


def _prep_body(coord_ref, tabsrc_ref, ih_ref, iw_ref, tab_ref):
    u = (coord_ref[...] / 100.0).astype(jnp.int32)
    ih_ref[...] = u[:, :, 0]
    iw_ref[...] = u[:, :, 1]
    tab_ref[...] = 0.1 * tabsrc_ref[...].T


_prep_call = pl.pallas_call(
    _prep_body,
    out_shape=(
        jax.ShapeDtypeStruct((B, L), jnp.int32),
        jax.ShapeDtypeStruct((B, L), jnp.int32),
        jax.ShapeDtypeStruct((384, DM), jnp.float32),
    ),
)


def _sc_body(x2, ih2, iw2, tab, out, *bufs):
    sets = [bufs[i * 9:(i + 1) * 9] for i in range(NBUF)]
    wid = lax.axis_index("s") * NC + lax.axis_index("c")
    tok0 = wid * TPW
    bi = tok0 // L
    l0 = tok0 % L

    def issue_in(c, S):
        ihv, iwv, xv, _, _, sem_x, sem_i, _, _ = S
        lc = l0 + c * CHUNK
        hx = pltpu.async_copy(x2.at[pl.ds(tok0 + c * CHUNK, CHUNK), :], xv, sem_x)
        h1 = pltpu.async_copy(ih2.at[bi, pl.ds(lc, CHUNK)], ihv, sem_i)
        h2 = pltpu.async_copy(iw2.at[bi, pl.ds(lc, CHUNK)], iwv, sem_i)
        return hx, h1, h2

    def issue_gathers(S):
        ihv, iwv, _, rowsh, rowsw, _, _, sem_g, _ = S
        gh = pltpu.async_copy(tab.at[ihv], rowsh, sem_g)
        gw = pltpu.async_copy(tab.at[iwv], rowsw, sem_g)
        return gh, gw

    def accumulate(S):
        _, _, xv, rowsh, rowsw, *_ = S

        def add_body(t, acc):
            for k in range(DM // LANES):
                plsc.addupdate(xv.at[t, pl.ds(k * LANES, LANES)],
                               rowsw[t, pl.ds(k * LANES, LANES)])
                plsc.addupdate(xv.at[t, pl.ds(DM + k * LANES, LANES)],
                               rowsh[t, pl.ds(k * LANES, LANES)])
            return acc

        lax.fori_loop(0, CHUNK, add_body, 0)

    h_in, h_g, h_out = {}, {}, {}
    for t in range(NCHUNK + 2):
        cA, cB, cC = t, t - 1, t - 2
        if cA < NCHUNK:
            if cA >= NBUF:
                h_out.pop(cA - NBUF).wait()
            h_in[cA] = issue_in(cA, sets[cA % NBUF])
        if 0 <= cB < NCHUNK:
            hx, h1, h2 = h_in.pop(cB)
            h1.wait()
            h2.wait()
            h_g[cB] = (hx,) + issue_gathers(sets[cB % NBUF])
        if 0 <= cC < NCHUNK:
            hx, gh, gw = h_g.pop(cC)
            hx.wait()
            gh.wait()
            gw.wait()
            S = sets[cC % NBUF]
            accumulate(S)
            h_out[cC] = pltpu.async_copy(
                S[2], out.at[pl.ds(tok0 + cC * CHUNK, CHUNK), :], S[8])
    for c in sorted(h_out):
        h_out.pop(c).wait()


def _buf_set():
    return [
        pltpu.VMEM((CHUNK,), jnp.int32),
        pltpu.VMEM((CHUNK,), jnp.int32),
        pltpu.VMEM((CHUNK, D_MODEL), jnp.float32),
        pltpu.VMEM((CHUNK, DM), jnp.float32),
        pltpu.VMEM((CHUNK, DM), jnp.float32),
        pltpu.SemaphoreType.DMA,
        pltpu.SemaphoreType.DMA,
        pltpu.SemaphoreType.DMA,
        pltpu.SemaphoreType.DMA,
    ]


_sc_call = pl.kernel(
    _sc_body,
    out_type=jax.ShapeDtypeStruct((NTOK, D_MODEL), jnp.float32),
    mesh=plsc.VectorSubcoreMesh(
        core_axis_name="c", subcore_axis_name="s",
        num_cores=NC, num_subcores=NS,
    ),
    scratch_types=_buf_set() + _buf_set() + _buf_set(),
)


@jax.jit
def kernel(x, coord, pe):
    ih2, iw2, tab = _prep_call(coord, pe[:DM, 0, :])
    out2 = _sc_call(x.reshape(NTOK, D_MODEL), ih2, iw2, tab)
    return out2.reshape(x.shape)


# speedup vs baseline: 1.2003x; 1.2003x over previous
import jax
import jax.numpy as jnp
from jax import lax
from jax.experimental import pallas as pl
from jax.experimental.pallas import tpu as pltpu
from jax.experimental.pallas import tpu_sc as plsc

D_MODEL = 256
NTOK = 16 * 2048
DM = D_MODEL // 2

NC, NS, LANES = 2, 16, 16
NW = NC * NS
TPW = NTOK // NW
CHUNK = 64
NCHUNK = TPW // CHUNK
SLOTS = 2 * CHUNK
NBUF = 3


def _tab_body(src, out):
    out[...] = 0.1 * src[...].T


_tab_call = pl.pallas_call(
    _tab_body,
    out_shape=jax.ShapeDtypeStruct((384, DM), jnp.float32),
)


def _sc_body(x2, cq, tab, out, *bufs):
    sets = [bufs[i * 8:(i + 1) * 8] for i in range(NBUF)]
    wid = lax.axis_index("s") * NC + lax.axis_index("c")
    tok0 = wid * TPW
    bi = tok0 // (NTOK // 16)
    p0 = (tok0 % (NTOK // 16)) * 2

    def issue_in(c, S):
        cv, _, xv, _, sem_x, sem_c, _, _ = S
        t0 = tok0 + c * CHUNK
        hx = pltpu.async_copy(x2.at[pl.ds(t0, CHUNK), :], xv, sem_x)
        hc = pltpu.async_copy(cq.at[bi, pl.ds(p0 + c * SLOTS, SLOTS)], cv, sem_c)
        return hx, hc

    def issue_gathers(S):
        cv, idxv, _, rowsv, _, _, sem_g, _ = S
        for g in range(SLOTS // LANES):
            sl = pl.ds(g * LANES, LANES)
            idxv[sl] = (cv[sl] / 100.0).astype(jnp.int32)
        return (pltpu.async_copy(tab.at[idxv], rowsv, sem_g),)

    def accumulate(S):
        xv, rowsv = S[2], S[3]

        def add_body(s, acc):
            cb = (1 - (s & 1)) * DM
            for k in range(DM // LANES):
                plsc.addupdate(xv.at[s >> 1, pl.ds(cb + k * LANES, LANES)],
                               rowsv[s, pl.ds(k * LANES, LANES)])
            return acc

        lax.fori_loop(0, SLOTS, add_body, 0)

    h_in, h_g, h_out = {}, {}, {}
    for t in range(NCHUNK + 2):
        cA, cB, cC = t, t - 1, t - 2
        if cA < NCHUNK:
            if cA >= NBUF:
                h_out.pop(cA - NBUF).wait()
            h_in[cA] = issue_in(cA, sets[cA % NBUF])
        if 0 <= cB < NCHUNK:
            hx, hc = h_in.pop(cB)
            hc.wait()
            h_g[cB] = (hx,) + issue_gathers(sets[cB % NBUF])
        if 0 <= cC < NCHUNK:
            hx, gr = h_g.pop(cC)
            hx.wait()
            gr.wait()
            S = sets[cC % NBUF]
            accumulate(S)
            h_out[cC] = pltpu.async_copy(
                S[2], out.at[pl.ds(tok0 + cC * CHUNK, CHUNK), :], S[7])
    for c in sorted(h_out):
        h_out.pop(c).wait()


def _buf_set():
    return [
        pltpu.VMEM((SLOTS,), jnp.float32),
        pltpu.VMEM((SLOTS,), jnp.int32),
        pltpu.VMEM((CHUNK, D_MODEL), jnp.float32),
        pltpu.VMEM((SLOTS, DM), jnp.float32),
        pltpu.SemaphoreType.DMA,
        pltpu.SemaphoreType.DMA,
        pltpu.SemaphoreType.DMA,
        pltpu.SemaphoreType.DMA,
    ]


_sc_call = pl.kernel(
    _sc_body,
    out_type=jax.ShapeDtypeStruct((NTOK, D_MODEL), jnp.float32),
    mesh=plsc.VectorSubcoreMesh(
        core_axis_name="c", subcore_axis_name="s",
        num_cores=NC, num_subcores=NS,
    ),
    scratch_types=_buf_set() + _buf_set() + _buf_set(),
)


@jax.jit
def kernel(x, coord, pe):
    tab = _tab_call(pe[:DM, 0, :])
    out2 = _sc_call(x.reshape(NTOK, D_MODEL), coord.reshape(16, NTOK // 8), tab)
    return out2.reshape(x.shape)


# speedup vs baseline: 1.2003x; 1.0001x over previous
import jax
import jax.numpy as jnp
from jax import lax
from jax.experimental import pallas as pl
from jax.experimental.pallas import tpu as pltpu
from jax.experimental.pallas import tpu_sc as plsc

D_MODEL = 256
NTOK = 16 * 2048
DM = D_MODEL // 2

NC, NS, LANES = 2, 16, 16
NW = NC * NS
TPW = NTOK // NW
CHUNK = 64
NCHUNK = TPW // CHUNK
SLOTS = 2 * CHUNK
NBUF = 3


def _tab_body(src, out):
    out[...] = 0.1 * src[...].T


_tab_call = pl.pallas_call(
    _tab_body,
    out_shape=jax.ShapeDtypeStruct((384, DM), jnp.float32),
)


def _sc_body(x2, cq, tab, out, *bufs):
    sets = [bufs[i * 8:(i + 1) * 8] for i in range(NBUF)]
    wid = lax.axis_index("s") * NC + lax.axis_index("c")
    tok0 = wid * TPW
    bi = tok0 // (NTOK // 16)
    p0 = (tok0 % (NTOK // 16)) * 2

    def issue_in(c, S):
        cv, _, xv, _, sem_x, sem_c, _, _ = S
        t0 = tok0 + c * CHUNK
        hx = pltpu.async_copy(x2.at[pl.ds(t0, CHUNK), :], xv, sem_x)
        hc = pltpu.async_copy(cq.at[bi, pl.ds(p0 + c * SLOTS, SLOTS)], cv, sem_c)
        return hx, hc

    def issue_gathers(S):
        cv, idxv, _, rowsv, _, _, sem_g, _ = S
        for g in range(SLOTS // LANES):
            sl = pl.ds(g * LANES, LANES)
            idxv[sl] = (cv[sl] / 100.0).astype(jnp.int32)
        return (pltpu.async_copy(tab.at[idxv], rowsv, sem_g),)

    def accumulate(S):
        xv, rowsv = S[2], S[3]

        @plsc.parallel_loop(0, CHUNK, unroll=4)
        def _add(t):
            for k in range(DM // LANES):
                plsc.addupdate(xv.at[t, pl.ds(k * LANES, LANES)],
                               rowsv[2 * t + 1, pl.ds(k * LANES, LANES)])
                plsc.addupdate(xv.at[t, pl.ds(DM + k * LANES, LANES)],
                               rowsv[2 * t, pl.ds(k * LANES, LANES)])

    h_in, h_g, h_out = {}, {}, {}
    for t in range(NCHUNK + 2):
        cA, cB, cC = t, t - 1, t - 2
        if cA < NCHUNK:
            if cA >= NBUF:
                h_out.pop(cA - NBUF).wait()
            h_in[cA] = issue_in(cA, sets[cA % NBUF])
        if 0 <= cB < NCHUNK:
            hx, hc = h_in.pop(cB)
            hc.wait()
            h_g[cB] = (hx,) + issue_gathers(sets[cB % NBUF])
        if 0 <= cC < NCHUNK:
            hx, gr = h_g.pop(cC)
            hx.wait()
            gr.wait()
            S = sets[cC % NBUF]
            accumulate(S)
            h_out[cC] = pltpu.async_copy(
                S[2], out.at[pl.ds(tok0 + cC * CHUNK, CHUNK), :], S[7])
    for c in sorted(h_out):
        h_out.pop(c).wait()


def _buf_set():
    return [
        pltpu.VMEM((SLOTS,), jnp.float32),
        pltpu.VMEM((SLOTS,), jnp.int32),
        pltpu.VMEM((CHUNK, D_MODEL), jnp.float32),
        pltpu.VMEM((SLOTS, DM), jnp.float32),
        pltpu.SemaphoreType.DMA,
        pltpu.SemaphoreType.DMA,
        pltpu.SemaphoreType.DMA,
        pltpu.SemaphoreType.DMA,
    ]


_sc_call = pl.kernel(
    _sc_body,
    out_type=jax.ShapeDtypeStruct((NTOK, D_MODEL), jnp.float32),
    mesh=plsc.VectorSubcoreMesh(
        core_axis_name="c", subcore_axis_name="s",
        num_cores=NC, num_subcores=NS,
    ),
    scratch_types=_buf_set() + _buf_set() + _buf_set(),
)


@jax.jit
def kernel(x, coord, pe):
    tab = _tab_call(pe[:DM, 0, :])
    out2 = _sc_call(x.reshape(NTOK, D_MODEL), coord.reshape(16, NTOK // 8), tab)
    return out2.reshape(x.shape)


# speedup vs baseline: 1.8928x; 1.5768x over previous
import jax
import jax.numpy as jnp
import numpy as np
from jax import lax
from jax.experimental import pallas as pl
from jax.experimental.pallas import tpu as pltpu
from jax.experimental.pallas import tpu_sc as plsc

D_MODEL = 256
NTOK = 16 * 2048
DM = D_MODEL // 2

NC, NS, LANES = 2, 16, 16
NW = NC * NS
TPW = NTOK // NW
CHUNK = 64
NCHUNK = TPW // CHUNK
SLOTS = 2 * CHUNK
GSUB = 128
NBUF = 3

_SRC = np.arange(128)
_SRC = 32 * (_SRC // 32) + (_SRC % 32) // 2 + 16 * (_SRC % 2)


def _tab_body(src, out):
    out[...] = 0.1 * src[...].T


_tab_call = pl.pallas_call(
    _tab_body,
    out_shape=jax.ShapeDtypeStruct((384, DM), jnp.float32),
)


def _sc_body(x2, cq, tab, out, tabs, *bufs):
    sets = [bufs[i * 8:(i + 1) * 8] for i in range(NBUF)]
    sid = lax.axis_index("s")
    wid = sid * NC + lax.axis_index("c")
    tok0 = wid * TPW
    bi = tok0 // (NTOK // 16)
    p0 = (tok0 % (NTOK // 16)) * 2

    @pl.when(sid == 0)
    def _stage():
        pltpu.sync_copy(tab, tabs)

    plsc.subcore_barrier()

    def issue_in(c, S):
        cv, _, xv, _, sem_x, sem_c, _, _ = S
        t0 = tok0 + c * CHUNK
        hx = pltpu.async_copy(x2.at[pl.ds(t0, CHUNK), :], xv, sem_x)
        hc = pltpu.async_copy(cq.at[bi, pl.ds(p0 + c * SLOTS, SLOTS)], cv, sem_c)
        return hx, hc

    def issue_gathers(S):
        cv, idxv, _, rowsv, _, _, sem_g, _ = S
        for g in range(SLOTS // LANES):
            sl = pl.ds(g * LANES, LANES)
            idxv[sl] = (cv[sl] / 100.0).astype(jnp.int32)
        return tuple(
            pltpu.async_copy(tabs.at[idxv.at[pl.ds(j * GSUB, GSUB)]],
                             rowsv.at[pl.ds(j * GSUB, GSUB), :], sem_g)
            for j in range(SLOTS // GSUB))

    def accumulate(S):
        xv, rowsv = S[2], S[3]

        @plsc.parallel_loop(0, CHUNK, unroll=4)
        def _add(t):
            for k in range(DM // LANES):
                plsc.addupdate(xv.at[t, pl.ds(k * LANES, LANES)],
                               rowsv[2 * t + 1, pl.ds(k * LANES, LANES)])
                plsc.addupdate(xv.at[t, pl.ds(DM + k * LANES, LANES)],
                               rowsv[2 * t, pl.ds(k * LANES, LANES)])

    h_in, h_g, h_out = {}, {}, {}
    for t in range(NCHUNK + 2):
        cA, cB, cC = t, t - 1, t - 2
        if cA < NCHUNK:
            if cA >= NBUF:
                h_out.pop(cA - NBUF).wait()
            h_in[cA] = issue_in(cA, sets[cA % NBUF])
        if 0 <= cB < NCHUNK:
            hx, hc = h_in.pop(cB)
            hc.wait()
            h_g[cB] = (hx,) + issue_gathers(sets[cB % NBUF])
        if 0 <= cC < NCHUNK:
            hx, *grs = h_g.pop(cC)
            hx.wait()
            for gr in grs:
                gr.wait()
            S = sets[cC % NBUF]
            accumulate(S)
            h_out[cC] = pltpu.async_copy(
                S[2], out.at[pl.ds(tok0 + cC * CHUNK, CHUNK), :], S[7])
    for c in sorted(h_out):
        h_out.pop(c).wait()


def _buf_set():
    return [
        pltpu.VMEM((SLOTS,), jnp.float32),
        pltpu.VMEM((SLOTS,), jnp.int32),
        pltpu.VMEM((CHUNK, D_MODEL), jnp.float32),
        pltpu.VMEM((SLOTS, DM), jnp.float32),
        pltpu.SemaphoreType.DMA,
        pltpu.SemaphoreType.DMA,
        pltpu.SemaphoreType.DMA,
        pltpu.SemaphoreType.DMA,
    ]


_sc_call = pl.kernel(
    _sc_body,
    out_type=jax.ShapeDtypeStruct((NTOK, D_MODEL), jnp.float32),
    mesh=plsc.VectorSubcoreMesh(
        core_axis_name="c", subcore_axis_name="s",
        num_cores=NC, num_subcores=NS,
    ),
    scratch_types=[pltpu.VMEM_SHARED((384, DM), jnp.float32)]
    + _buf_set() + _buf_set() + _buf_set(),
)


@jax.jit
def kernel(x, coord, pe):
    tab = _tab_call(pe[:DM, 0, :])
    out2 = _sc_call(x.reshape(NTOK, D_MODEL), coord.reshape(16, NTOK // 8),
                    tab)
    return out2.reshape(x.shape)


# speedup vs baseline: 2.2610x; 1.1946x over previous
import jax
import jax.numpy as jnp
import numpy as np
from jax import lax
from jax.experimental import pallas as pl
from jax.experimental.pallas import tpu as pltpu
from jax.experimental.pallas import tpu_sc as plsc

D_MODEL = 256
NTOK = 16 * 2048
DM = D_MODEL // 2

NC, NS, LANES = 2, 16, 16
NW = NC * NS
TPW = NTOK // NW
CHUNK = 64
NCHUNK = TPW // CHUNK
SLOTS = 2 * CHUNK
GSUB = 128
NBUF = 3

_SRC = np.arange(128)
_SRC = 32 * (_SRC // 32) + (_SRC % 32) // 2 + 16 * (_SRC % 2)


def _tab_body(src, out):
    out[...] = 0.1 * src[...].T


_tab_call = pl.pallas_call(
    _tab_body,
    out_shape=jax.ShapeDtypeStruct((384, DM), jnp.float32),
)


def _sc_body(x2, cq, tab, out, tabs, *bufs):
    sets = [bufs[i * 9:(i + 1) * 9] for i in range(NBUF)]
    sid = lax.axis_index("s")
    wid = sid * NC + lax.axis_index("c")
    tok0 = wid * TPW
    bi = tok0 // (NTOK // 16)
    p0 = (tok0 % (NTOK // 16)) * 2

    @pl.when(sid == 0)
    def _stage():
        pltpu.sync_copy(tab, tabs)

    plsc.subcore_barrier()

    lane = lax.iota(jnp.int32, LANES)
    lo8 = lane < 8
    podd = (2 * lane + 1) % LANES
    peven = (2 * lane) % LANES
    dnums = lax.GatherDimensionNumbers(
        offset_dims=(), collapsed_slice_dims=(0,), start_index_map=(0,))

    def _vgather(v, p):
        return lax.gather(v, p[:, None], dimension_numbers=dnums,
                          slice_sizes=(1,),
                          mode=lax.GatherScatterMode.PROMISE_IN_BOUNDS)

    def issue_in(c, S):
        cv, _, _, xvw, xvh, sem_x, sem_c, _, _ = S
        t0 = tok0 + c * CHUNK
        hx1 = pltpu.async_copy(x2.at[pl.ds(t0, CHUNK), pl.ds(0, DM)], xvw, sem_x)
        hx2 = pltpu.async_copy(x2.at[pl.ds(t0, CHUNK), pl.ds(DM, DM)], xvh, sem_x)
        hc = pltpu.async_copy(cq.at[bi, pl.ds(p0 + c * SLOTS, SLOTS)], cv, sem_c)
        return hx1, hx2, hc

    def issue_adds(S):
        cv, ihv, iwv, xvw, xvh, _, _, sem_g, _ = S
        for m in range(CHUNK // LANES):
            v0 = cv[pl.ds(2 * m * LANES, LANES)]
            v1 = cv[pl.ds((2 * m + 1) * LANES, LANES)]
            sl = pl.ds(m * LANES, LANES)
            hsel = jnp.where(lo8, _vgather(v0, peven), _vgather(v1, peven))
            wsel = jnp.where(lo8, _vgather(v0, podd), _vgather(v1, podd))
            ihv[sl] = (hsel / 100.0).astype(jnp.int32)
            iwv[sl] = (wsel / 100.0).astype(jnp.int32)
        gw = pltpu.async_copy(tabs.at[iwv], xvw, sem_g, add=True)
        gh = pltpu.async_copy(tabs.at[ihv], xvh, sem_g, add=True)
        return gw, gh

    h_in, h_g, h_out = {}, {}, {}
    for t in range(NCHUNK + 2):
        cA, cB, cC = t, t - 1, t - 2
        if cA < NCHUNK:
            if cA >= NBUF:
                for ho in h_out.pop(cA - NBUF):
                    ho.wait()
            h_in[cA] = issue_in(cA, sets[cA % NBUF])
        if 0 <= cB < NCHUNK:
            hx1, hx2, hc = h_in.pop(cB)
            hx1.wait()
            hx2.wait()
            hc.wait()
            h_g[cB] = issue_adds(sets[cB % NBUF])
        if 0 <= cC < NCHUNK:
            for gr in h_g.pop(cC):
                gr.wait()
            S = sets[cC % NBUF]
            t0 = tok0 + cC * CHUNK
            h_out[cC] = (
                pltpu.async_copy(
                    S[3], out.at[pl.ds(t0, CHUNK), pl.ds(0, DM)], S[8]),
                pltpu.async_copy(
                    S[4], out.at[pl.ds(t0, CHUNK), pl.ds(DM, DM)], S[8]),
            )
    for c in sorted(h_out):
        for ho in h_out.pop(c):
            ho.wait()


def _buf_set():
    return [
        pltpu.VMEM((SLOTS,), jnp.float32),
        pltpu.VMEM((CHUNK,), jnp.int32),
        pltpu.VMEM((CHUNK,), jnp.int32),
        pltpu.VMEM((CHUNK, DM), jnp.float32),
        pltpu.VMEM((CHUNK, DM), jnp.float32),
        pltpu.SemaphoreType.DMA,
        pltpu.SemaphoreType.DMA,
        pltpu.SemaphoreType.DMA,
        pltpu.SemaphoreType.DMA,
    ]


_sc_call = pl.kernel(
    _sc_body,
    out_type=jax.ShapeDtypeStruct((NTOK, D_MODEL), jnp.float32),
    mesh=plsc.VectorSubcoreMesh(
        core_axis_name="c", subcore_axis_name="s",
        num_cores=NC, num_subcores=NS,
    ),
    scratch_types=[pltpu.VMEM_SHARED((384, DM), jnp.float32)]
    + _buf_set() + _buf_set() + _buf_set(),
)


@jax.jit
def kernel(x, coord, pe):
    tab = _tab_call(pe[:DM, 0, :])
    out2 = _sc_call(x.reshape(NTOK, D_MODEL), coord.reshape(16, NTOK // 8),
                    tab)
    return out2.reshape(x.shape)


# speedup vs baseline: 2.2964x; 1.0156x over previous
import jax
import jax.numpy as jnp
import numpy as np
from jax import lax
from jax.experimental import pallas as pl
from jax.experimental.pallas import tpu as pltpu
from jax.experimental.pallas import tpu_sc as plsc

D_MODEL = 256
NTOK = 16 * 2048
DM = D_MODEL // 2

NC, NS, LANES = 2, 16, 16
NW = NC * NS
TPW = NTOK // NW
CHUNK = 128
NCHUNK = TPW // CHUNK
SLOTS = 2 * CHUNK
GSUB = 128
NBUF = 3

_SRC = np.arange(128)
_SRC = 32 * (_SRC // 32) + (_SRC % 32) // 2 + 16 * (_SRC % 2)


def _tab_body(src, out):
    out[...] = 0.1 * src[...].T


_tab_call = pl.pallas_call(
    _tab_body,
    out_shape=jax.ShapeDtypeStruct((384, DM), jnp.float32),
)


def _sc_body(x2, cq, tab, out, tabs, *bufs):
    sets = [bufs[i * 9:(i + 1) * 9] for i in range(NBUF)]
    sid = lax.axis_index("s")
    wid = sid * NC + lax.axis_index("c")
    tok0 = wid * TPW
    bi = tok0 // (NTOK // 16)
    p0 = (tok0 % (NTOK // 16)) * 2

    @pl.when(sid == 0)
    def _stage():
        pltpu.sync_copy(tab, tabs)

    plsc.subcore_barrier()

    lane = lax.iota(jnp.int32, LANES)
    lo8 = lane < 8
    podd = (2 * lane + 1) % LANES
    peven = (2 * lane) % LANES
    dnums = lax.GatherDimensionNumbers(
        offset_dims=(), collapsed_slice_dims=(0,), start_index_map=(0,))

    def _vgather(v, p):
        return lax.gather(v, p[:, None], dimension_numbers=dnums,
                          slice_sizes=(1,),
                          mode=lax.GatherScatterMode.PROMISE_IN_BOUNDS)

    def issue_in(c, S):
        cv, _, _, xvw, xvh, sem_x, sem_c, _, _ = S
        t0 = tok0 + c * CHUNK
        hx1 = pltpu.async_copy(x2.at[pl.ds(t0, CHUNK), pl.ds(0, DM)], xvw, sem_x)
        hx2 = pltpu.async_copy(x2.at[pl.ds(t0, CHUNK), pl.ds(DM, DM)], xvh, sem_x)
        hc = pltpu.async_copy(cq.at[bi, pl.ds(p0 + c * SLOTS, SLOTS)], cv, sem_c)
        return hx1, hx2, hc

    def issue_adds(S):
        cv, ihv, iwv, xvw, xvh, _, _, sem_g, _ = S
        for m in range(CHUNK // LANES):
            v0 = cv[pl.ds(2 * m * LANES, LANES)]
            v1 = cv[pl.ds((2 * m + 1) * LANES, LANES)]
            sl = pl.ds(m * LANES, LANES)
            hsel = jnp.where(lo8, _vgather(v0, peven), _vgather(v1, peven))
            wsel = jnp.where(lo8, _vgather(v0, podd), _vgather(v1, podd))
            ihv[sl] = (hsel / 100.0).astype(jnp.int32)
            iwv[sl] = (wsel / 100.0).astype(jnp.int32)
        gw = pltpu.async_copy(tabs.at[iwv], xvw, sem_g, add=True)
        gh = pltpu.async_copy(tabs.at[ihv], xvh, sem_g, add=True)
        return gw, gh

    h_in, h_g, h_out = {}, {}, {}
    for t in range(NCHUNK + 2):
        cA, cB, cC = t, t - 1, t - 2
        if cA < NCHUNK:
            if cA >= NBUF:
                for ho in h_out.pop(cA - NBUF):
                    ho.wait()
            h_in[cA] = issue_in(cA, sets[cA % NBUF])
        if 0 <= cB < NCHUNK:
            hx1, hx2, hc = h_in.pop(cB)
            hx1.wait()
            hx2.wait()
            hc.wait()
            h_g[cB] = issue_adds(sets[cB % NBUF])
        if 0 <= cC < NCHUNK:
            for gr in h_g.pop(cC):
                gr.wait()
            S = sets[cC % NBUF]
            t0 = tok0 + cC * CHUNK
            h_out[cC] = (
                pltpu.async_copy(
                    S[3], out.at[pl.ds(t0, CHUNK), pl.ds(0, DM)], S[8]),
                pltpu.async_copy(
                    S[4], out.at[pl.ds(t0, CHUNK), pl.ds(DM, DM)], S[8]),
            )
    for c in sorted(h_out):
        for ho in h_out.pop(c):
            ho.wait()


def _buf_set():
    return [
        pltpu.VMEM((SLOTS,), jnp.float32),
        pltpu.VMEM((CHUNK,), jnp.int32),
        pltpu.VMEM((CHUNK,), jnp.int32),
        pltpu.VMEM((CHUNK, DM), jnp.float32),
        pltpu.VMEM((CHUNK, DM), jnp.float32),
        pltpu.SemaphoreType.DMA,
        pltpu.SemaphoreType.DMA,
        pltpu.SemaphoreType.DMA,
        pltpu.SemaphoreType.DMA,
    ]


_sc_call = pl.kernel(
    _sc_body,
    out_type=jax.ShapeDtypeStruct((NTOK, D_MODEL), jnp.float32),
    mesh=plsc.VectorSubcoreMesh(
        core_axis_name="c", subcore_axis_name="s",
        num_cores=NC, num_subcores=NS,
    ),
    scratch_types=[pltpu.VMEM_SHARED((384, DM), jnp.float32)]
    + _buf_set() + _buf_set() + _buf_set(),
)


@jax.jit
def kernel(x, coord, pe):
    tab = _tab_call(pe[:DM, 0, :])
    out2 = _sc_call(x.reshape(NTOK, D_MODEL), coord.reshape(16, NTOK // 8),
                    tab)
    return out2.reshape(x.shape)


# speedup vs baseline: 2.4444x; 1.0644x over previous
import jax
import jax.numpy as jnp
import numpy as np
from jax import lax
from jax.experimental import pallas as pl
from jax.experimental.pallas import tpu as pltpu
from jax.experimental.pallas import tpu_sc as plsc

D_MODEL = 256
NTOK = 16 * 2048
DM = D_MODEL // 2

NC, NS, LANES = 2, 16, 16
NW = NC * NS
TPW = NTOK // NW
CHUNK = 128
NCHUNK = TPW // CHUNK
SLOTS = 2 * CHUNK
GSUB = 128
NBUF = 3

_SRC = np.arange(128)
_SRC = 32 * (_SRC // 32) + (_SRC % 32) // 2 + 16 * (_SRC % 2)


def _tab_body(src, out):
    out[...] = 0.1 * src[:, 0, :].T


_tab_call = pl.pallas_call(
    _tab_body,
    grid=(1,),
    in_specs=[pl.BlockSpec((DM, 8, 384), lambda i: (0, 0, 0))],
    out_specs=pl.BlockSpec((384, DM), lambda i: (0, 0)),
    out_shape=jax.ShapeDtypeStruct((384, DM), jnp.float32),
)


def _sc_body(x2, cq, tab, out, tabs, *bufs):
    sets = [bufs[i * 9:(i + 1) * 9] for i in range(NBUF)]
    sid = lax.axis_index("s")
    wid = sid * NC + lax.axis_index("c")
    tok0 = wid * TPW
    bi = tok0 // (NTOK // 16)
    p0 = (tok0 % (NTOK // 16)) * 2

    @pl.when(sid == 0)
    def _stage():
        pltpu.sync_copy(tab, tabs)

    plsc.subcore_barrier()

    lane = lax.iota(jnp.int32, LANES)
    lo8 = lane < 8
    podd = (2 * lane + 1) % LANES
    peven = (2 * lane) % LANES
    dnums = lax.GatherDimensionNumbers(
        offset_dims=(), collapsed_slice_dims=(0,), start_index_map=(0,))

    def _vgather(v, p):
        return lax.gather(v, p[:, None], dimension_numbers=dnums,
                          slice_sizes=(1,),
                          mode=lax.GatherScatterMode.PROMISE_IN_BOUNDS)

    def issue_in(c, S):
        cv, _, _, xvw, xvh, sem_x, sem_c, _, _ = S
        t0 = tok0 + c * CHUNK
        hx1 = pltpu.async_copy(x2.at[pl.ds(t0, CHUNK), pl.ds(0, DM)], xvw, sem_x)
        hx2 = pltpu.async_copy(x2.at[pl.ds(t0, CHUNK), pl.ds(DM, DM)], xvh, sem_x)
        hc = pltpu.async_copy(cq.at[bi, pl.ds(p0 + c * SLOTS, SLOTS)], cv, sem_c)
        return hx1, hx2, hc

    def issue_adds(S):
        cv, ihv, iwv, xvw, xvh, _, _, sem_g, _ = S
        for m in range(CHUNK // LANES):
            v0 = cv[pl.ds(2 * m * LANES, LANES)]
            v1 = cv[pl.ds((2 * m + 1) * LANES, LANES)]
            sl = pl.ds(m * LANES, LANES)
            hsel = jnp.where(lo8, _vgather(v0, peven), _vgather(v1, peven))
            wsel = jnp.where(lo8, _vgather(v0, podd), _vgather(v1, podd))
            ihv[sl] = (hsel / 100.0).astype(jnp.int32)
            iwv[sl] = (wsel / 100.0).astype(jnp.int32)
        gw = pltpu.async_copy(tabs.at[iwv], xvw, sem_g, add=True)
        gh = pltpu.async_copy(tabs.at[ihv], xvh, sem_g, add=True)
        return gw, gh

    h_in, h_g, h_out = {}, {}, {}
    for t in range(NCHUNK + 2):
        cA, cB, cC = t, t - 1, t - 2
        if cA < NCHUNK:
            if cA >= NBUF:
                for ho in h_out.pop(cA - NBUF):
                    ho.wait()
            h_in[cA] = issue_in(cA, sets[cA % NBUF])
        if 0 <= cB < NCHUNK:
            hx1, hx2, hc = h_in.pop(cB)
            hx1.wait()
            hx2.wait()
            hc.wait()
            h_g[cB] = issue_adds(sets[cB % NBUF])
        if 0 <= cC < NCHUNK:
            for gr in h_g.pop(cC):
                gr.wait()
            S = sets[cC % NBUF]
            t0 = tok0 + cC * CHUNK
            h_out[cC] = (
                pltpu.async_copy(
                    S[3], out.at[pl.ds(t0, CHUNK), pl.ds(0, DM)], S[8]),
                pltpu.async_copy(
                    S[4], out.at[pl.ds(t0, CHUNK), pl.ds(DM, DM)], S[8]),
            )
    for c in sorted(h_out):
        for ho in h_out.pop(c):
            ho.wait()


def _buf_set():
    return [
        pltpu.VMEM((SLOTS,), jnp.float32),
        pltpu.VMEM((CHUNK,), jnp.int32),
        pltpu.VMEM((CHUNK,), jnp.int32),
        pltpu.VMEM((CHUNK, DM), jnp.float32),
        pltpu.VMEM((CHUNK, DM), jnp.float32),
        pltpu.SemaphoreType.DMA,
        pltpu.SemaphoreType.DMA,
        pltpu.SemaphoreType.DMA,
        pltpu.SemaphoreType.DMA,
    ]


_sc_call = pl.kernel(
    _sc_body,
    out_type=jax.ShapeDtypeStruct((NTOK, D_MODEL), jnp.float32),
    mesh=plsc.VectorSubcoreMesh(
        core_axis_name="c", subcore_axis_name="s",
        num_cores=NC, num_subcores=NS,
    ),
    scratch_types=[pltpu.VMEM_SHARED((384, DM), jnp.float32)]
    + _buf_set() + _buf_set() + _buf_set(),
)


@jax.jit
def kernel(x, coord, pe):
    tab = _tab_call(pe)
    out2 = _sc_call(x.reshape(NTOK, D_MODEL), coord.reshape(16, NTOK // 8),
                    tab)
    return out2.reshape(x.shape)


# speedup vs baseline: 2.4493x; 1.0020x over previous
import jax
import jax.numpy as jnp
from jax import lax
from jax.experimental import pallas as pl
from jax.experimental.pallas import tpu as pltpu
from jax.experimental.pallas import tpu_sc as plsc

D_MODEL = 256
NTOK = 16 * 2048
DM = D_MODEL // 2

NC, NS, LANES = 2, 16, 16
NW = NC * NS
TPW = NTOK // NW
CHUNK = 128
NCHUNK = TPW // CHUNK
SLOTS = 2 * CHUNK
NBUF = 3


def _tab_body(src, out):
    out[...] = 0.1 * src[:, 0, :].T


_tab_call = pl.pallas_call(
    _tab_body,
    grid=(1,),
    in_specs=[pl.BlockSpec((DM, 8, 384), lambda i: (0, 0, 0))],
    out_specs=pl.BlockSpec((384, DM), lambda i: (0, 0)),
    out_shape=jax.ShapeDtypeStruct((384, DM), jnp.float32),
)


def _sc_body(x2, cq, tab, out, tabs, *bufs):
    sets = [bufs[i * 9:(i + 1) * 9] for i in range(NBUF)]
    sid = lax.axis_index("s")
    wid = sid * NC + lax.axis_index("c")
    tok0 = wid * TPW
    bi = tok0 // (NTOK // 16)
    p0 = (tok0 % (NTOK // 16)) * 2

    @pl.when(sid == 0)
    def _stage():
        pltpu.sync_copy(tab, tabs)

    plsc.subcore_barrier()

    lane = lax.iota(jnp.int32, LANES)
    lo8 = lane < 8
    podd = (2 * lane + 1) % LANES
    peven = (2 * lane) % LANES
    dnums = lax.GatherDimensionNumbers(
        offset_dims=(), collapsed_slice_dims=(0,), start_index_map=(0,))

    def _vgather(v, p):
        return lax.gather(v, p[:, None], dimension_numbers=dnums,
                          slice_sizes=(1,),
                          mode=lax.GatherScatterMode.PROMISE_IN_BOUNDS)

    def issue_in(c, S):
        cv, _, _, xvw, xvh, sem_x, sem_c, _, _ = S
        t0 = tok0 + c * CHUNK
        hx1 = pltpu.async_copy(x2.at[pl.ds(t0, CHUNK), pl.ds(0, DM)], xvw, sem_x)
        hx2 = pltpu.async_copy(x2.at[pl.ds(t0, CHUNK), pl.ds(DM, DM)], xvh, sem_x)
        hc = pltpu.async_copy(cq.at[bi, pl.ds(p0 + c * SLOTS, SLOTS)], cv, sem_c)
        return hx1, hx2, hc

    def issue_adds(S):
        cv, ihv, iwv, xvw, xvh, _, _, sem_g, _ = S
        for m in range(CHUNK // LANES):
            v0 = cv[pl.ds(2 * m * LANES, LANES)]
            v1 = cv[pl.ds((2 * m + 1) * LANES, LANES)]
            sl = pl.ds(m * LANES, LANES)
            hsel = jnp.where(lo8, _vgather(v0, peven), _vgather(v1, peven))
            wsel = jnp.where(lo8, _vgather(v0, podd), _vgather(v1, podd))
            ihv[sl] = (hsel / 100.0).astype(jnp.int32)
            iwv[sl] = (wsel / 100.0).astype(jnp.int32)
        gw = pltpu.async_copy(tabs.at[iwv], xvw, sem_g, add=True)
        gh = pltpu.async_copy(tabs.at[ihv], xvh, sem_g, add=True)
        return gw, gh

    h_in, h_g, h_out = {}, {}, {}
    for t in range(NCHUNK + 2):
        cA, cB, cC = t, t - 1, t - 2
        if cA < NCHUNK:
            if cA >= NBUF:
                for ho in h_out.pop(cA - NBUF):
                    ho.wait()
            h_in[cA] = issue_in(cA, sets[cA % NBUF])
        if 0 <= cB < NCHUNK:
            hx1, hx2, hc = h_in.pop(cB)
            hx1.wait()
            hx2.wait()
            hc.wait()
            h_g[cB] = issue_adds(sets[cB % NBUF])
        if 0 <= cC < NCHUNK:
            for gr in h_g.pop(cC):
                gr.wait()
            S = sets[cC % NBUF]
            t0 = tok0 + cC * CHUNK
            h_out[cC] = (
                pltpu.async_copy(
                    S[3], out.at[pl.ds(t0, CHUNK), pl.ds(0, DM)], S[8]),
                pltpu.async_copy(
                    S[4], out.at[pl.ds(t0, CHUNK), pl.ds(DM, DM)], S[8]),
            )
    for c in sorted(h_out):
        for ho in h_out.pop(c):
            ho.wait()


def _buf_set():
    return [
        pltpu.VMEM((SLOTS,), jnp.float32),
        pltpu.VMEM((CHUNK,), jnp.int32),
        pltpu.VMEM((CHUNK,), jnp.int32),
        pltpu.VMEM((CHUNK, DM), jnp.float32),
        pltpu.VMEM((CHUNK, DM), jnp.float32),
        pltpu.SemaphoreType.DMA,
        pltpu.SemaphoreType.DMA,
        pltpu.SemaphoreType.DMA,
        pltpu.SemaphoreType.DMA,
    ]


_sc_call = pl.kernel(
    _sc_body,
    out_type=jax.ShapeDtypeStruct((NTOK, D_MODEL), jnp.float32),
    mesh=plsc.VectorSubcoreMesh(
        core_axis_name="c", subcore_axis_name="s",
        num_cores=NC, num_subcores=NS,
    ),
    scratch_types=[pltpu.VMEM_SHARED((384, DM), jnp.float32)]
    + _buf_set() + _buf_set() + _buf_set(),
)


@jax.jit
def kernel(x, coord, pe):
    tab = _tab_call(pe)
    out2 = _sc_call(x.reshape(NTOK, D_MODEL), coord.reshape(16, NTOK // 8),
                    tab)
    return out2.reshape(x.shape)
